# baseline (speedup 1.0000x reference)
import functools

import jax
import jax.numpy as jnp
from jax import lax
from jax.experimental import pallas as pl
from jax.experimental.pallas import tpu as pltpu
from jax.experimental.pallas import tpu_sc as plsc


def _tc_add_body(x_ref, t_ref, o_ref):
    o_ref[...] = x_ref[...] + t_ref[...][None, :, :]


def _tc_kernel(x, table):
    B, S, D = x.shape
    bs = 512
    return pl.pallas_call(
        _tc_add_body,
        grid=(S // bs,),
        in_specs=[
            pl.BlockSpec((B, bs, D), lambda i: (0, i, 0)),
            pl.BlockSpec((bs, D), lambda i: (i, 0)),
        ],
        out_specs=pl.BlockSpec((B, bs, D), lambda i: (0, i, 0)),
        out_shape=jax.ShapeDtypeStruct(x.shape, x.dtype),
    )(x, table)


def _sc_make(N, T, NC, NS, CH):
    NW = NC * NS
    EW = N // NW
    WPB = T // EW
    n_chunks = EW // CH
    mesh = plsc.VectorSubcoreMesh(core_axis_name="c", subcore_axis_name="s")

    @functools.partial(
        pl.kernel,
        out_type=jax.ShapeDtypeStruct((N,), jnp.float32),
        mesh=mesh,
        scratch_types=[
            pltpu.VMEM((CH,), jnp.float32),
            pltpu.VMEM((CH,), jnp.float32),
            pltpu.SemaphoreType.DMA,
            pltpu.SemaphoreType.DMA,
        ],
    )
    def k(x_hbm, t_hbm, o_hbm, xbuf, tbuf, semx, semt):
        wid = lax.axis_index("s") * NC + lax.axis_index("c")
        xbase = wid * EW
        tbase = (wid % WPB) * EW

        def body(i, carry):
            off = i * CH
            cx = pltpu.make_async_copy(
                x_hbm.at[pl.ds(xbase + off, CH)], xbuf, semx)
            ct = pltpu.make_async_copy(
                t_hbm.at[pl.ds(tbase + off, CH)], tbuf, semt)
            cx.start()
            ct.start()
            cx.wait()
            ct.wait()

            def inner(j, c):
                sl = pl.ds(j * 16, 16)
                xbuf[sl] = xbuf[sl] + tbuf[sl]
                return c

            lax.fori_loop(0, CH // 16, inner, 0, unroll=8)
            pltpu.sync_copy(xbuf, o_hbm.at[pl.ds(xbase + off, CH)])
            return carry

        lax.fori_loop(0, n_chunks, body, 0)

    return k


def _sc_kernel(x, table):
    B, S, D = x.shape
    N = B * S * D
    T = S * D
    info = plsc.get_sparse_core_info()
    NC, NS = info.num_cores, info.num_subcores
    out = _sc_make(N, T, NC, NS, 32768)(x.reshape(N), table.reshape(T))
    return out.reshape(B, S, D)


def kernel(x, table):
    return _sc_kernel(x, table)


# speedup vs baseline: 6.8487x; 6.8487x over previous
import functools

import jax
import jax.numpy as jnp
from jax import lax
from jax.experimental import pallas as pl
from jax.experimental.pallas import tpu as pltpu
from jax.experimental.pallas import tpu_sc as plsc


def _tc_add_body(x_ref, t_ref, o_ref):
    o_ref[...] = x_ref[...] + t_ref[...][None, :, :]


def _tc_kernel(x, table):
    B, S, D = x.shape
    bs = 512
    return pl.pallas_call(
        _tc_add_body,
        grid=(S // bs,),
        in_specs=[
            pl.BlockSpec((B, bs, D), lambda i: (0, i, 0)),
            pl.BlockSpec((bs, D), lambda i: (i, 0)),
        ],
        out_specs=pl.BlockSpec((B, bs, D), lambda i: (0, i, 0)),
        out_shape=jax.ShapeDtypeStruct(x.shape, x.dtype),
    )(x, table)


def _tc_add_body2d(x_ref, t_ref, o_ref):
    o_ref[...] = x_ref[...] + t_ref[...]


def _tc_kernel2d(x, table):
    B, S, D = x.shape
    bs = 512
    x2 = x.reshape(B * S, D)
    nsb = S // bs
    out = pl.pallas_call(
        _tc_add_body2d,
        grid=(nsb, B),
        in_specs=[
            pl.BlockSpec((bs, D), lambda i, b: (b * nsb + i, 0)),
            pl.BlockSpec((bs, D), lambda i, b: (i, 0)),
        ],
        out_specs=pl.BlockSpec((bs, D), lambda i, b: (b * nsb + i, 0)),
        out_shape=jax.ShapeDtypeStruct(x2.shape, x2.dtype),
    )(x2, table)
    return out.reshape(B, S, D)


def _sc_make(N, T, NC, NS, CH):
    NW = NC * NS
    EW = N // NW
    WPB = T // EW
    n_chunks = EW // CH
    mesh = plsc.VectorSubcoreMesh(core_axis_name="c", subcore_axis_name="s")

    @functools.partial(
        pl.kernel,
        out_type=jax.ShapeDtypeStruct((N,), jnp.float32),
        mesh=mesh,
        scratch_types=[
            pltpu.VMEM((CH,), jnp.float32),
            pltpu.VMEM((CH,), jnp.float32),
            pltpu.SemaphoreType.DMA,
            pltpu.SemaphoreType.DMA,
        ],
    )
    def k(x_hbm, t_hbm, o_hbm, xbuf, tbuf, semx, semt):
        wid = lax.axis_index("s") * NC + lax.axis_index("c")
        xbase = wid * EW
        tbase = (wid % WPB) * EW

        def body(i, carry):
            off = i * CH
            cx = pltpu.make_async_copy(
                x_hbm.at[pl.ds(xbase + off, CH)], xbuf, semx)
            ct = pltpu.make_async_copy(
                t_hbm.at[pl.ds(tbase + off, CH)], tbuf, semt)
            cx.start()
            ct.start()
            cx.wait()
            ct.wait()

            def inner(j, c):
                sl = pl.ds(j * 16, 16)
                xbuf[sl] = xbuf[sl] + tbuf[sl]
                return c

            lax.fori_loop(0, CH // 16, inner, 0, unroll=8)
            pltpu.sync_copy(xbuf, o_hbm.at[pl.ds(xbase + off, CH)])
            return carry

        lax.fori_loop(0, n_chunks, body, 0)

    return k


def _sc_kernel(x, table):
    B, S, D = x.shape
    N = B * S * D
    T = S * D
    info = plsc.get_sparse_core_info()
    NC, NS = info.num_cores, info.num_subcores
    out = _sc_make(N, T, NC, NS, 32768)(x.reshape(N), table.reshape(T))
    return out.reshape(B, S, D)


def kernel(x, table):
    return _tc_kernel2d(x, table)
